# baseline (speedup 1.0000x reference)
import functools

import jax
import jax.numpy as jnp
from jax import lax
from jax.experimental import pallas as pl
from jax.experimental.pallas import tpu as pltpu
from jax.experimental.pallas import tpu_sc as plsc

NUM_HIDDEN = 32
BATCH = 16384
NC = 2
NS = 16
NW = NC * NS
B_PER_W = BATCH // NW
HALF = B_PER_W // 2
L = 16
NSEM = 8


def _scalar(vec, j):
    return jnp.squeeze(lax.slice(vec, (j,), (j + 1,)))


def _make_sc_kernel():
    mesh = plsc.VectorSubcoreMesh(core_axis_name="c", subcore_axis_name="s")

    @functools.partial(
        pl.kernel,
        mesh=mesh,
        compiler_params=pltpu.CompilerParams(needs_layout_passes=False),
        out_type=jax.ShapeDtypeStruct((BATCH,), jnp.float32),
        scratch_types=[
            pltpu.VMEM((B_PER_W,), jnp.int32),
            pltpu.VMEM((B_PER_W,), jnp.int32),
            pltpu.VMEM((HALF, NUM_HIDDEN), jnp.float32),
            pltpu.VMEM((HALF, NUM_HIDDEN), jnp.float32),
            pltpu.VMEM((B_PER_W,), jnp.float32),
            [pltpu.SemaphoreType.DMA] * NSEM,
            [pltpu.SemaphoreType.DMA] * NSEM,
        ],
    )
    def sc_kernel(uidx_hbm, iidx_hbm, user_hbm, item_hbm, out_hbm,
                  uidx_v, iidx_v, urows_v, irows_v, out_v,
                  sems_u, sems_i):
        wid = lax.axis_index("s") * NC + lax.axis_index("c")
        base = wid * B_PER_W
        row_iota = lax.iota(jnp.int32, L)

        pltpu.sync_copy(uidx_hbm.at[pl.ds(base, B_PER_W)], uidx_v)
        pltpu.sync_copy(iidx_hbm.at[pl.ds(base, B_PER_W)], iidx_v)

        def half_body(h, carry):
            hbase = h * HALF

            @plsc.parallel_loop(0, HALF, step=L)
            def _fetch(g):
                uvec = uidx_v[pl.ds(hbase + g, L)]
                ivec = iidx_v[pl.ds(hbase + g, L)]
                for j in range(L):
                    pltpu.async_copy(
                        user_hbm.at[pl.ds(_scalar(uvec, j), 1)],
                        urows_v.at[pl.ds(g + j, 1)], sems_u[j % NSEM])
                    pltpu.async_copy(
                        item_hbm.at[pl.ds(_scalar(ivec, j), 1)],
                        irows_v.at[pl.ds(g + j, 1)], sems_i[j % NSEM])

            rows_per_sem = HALF // NSEM
            for k in range(NSEM):
                pltpu.make_async_copy(
                    user_hbm.at[pl.ds(0, rows_per_sem)],
                    urows_v.at[pl.ds(0, rows_per_sem)], sems_u[k]).wait()
                pltpu.make_async_copy(
                    item_hbm.at[pl.ds(0, rows_per_sem)],
                    irows_v.at[pl.ds(0, rows_per_sem)], sems_i[k]).wait()

            def group_body(g, c):
                rows = g * L + row_iota
                acc = jnp.zeros((L,), jnp.float32)
                for col_h in range(NUM_HIDDEN):
                    col = jnp.full((L,), col_h, jnp.int32)
                    u = plsc.load_gather(urows_v, [rows, col])
                    v = plsc.load_gather(irows_v, [rows, col])
                    acc = acc + u * v
                out_v[pl.ds(hbase + g * L, L)] = acc
                return c

            lax.fori_loop(0, HALF // L, group_body, 0)
            return carry

        lax.fori_loop(0, 2, half_body, 0)
        pltpu.sync_copy(out_v, out_hbm.at[pl.ds(base, B_PER_W)])

    return sc_kernel


_SC_KERNEL = _make_sc_kernel()


@jax.jit
def kernel(indices, ratings, user_table, item_table):
    idx = indices.astype(jnp.int32)
    pred = _SC_KERNEL(idx[0], idx[1], user_table, item_table)
    return (pred, ratings)
